# baseline (speedup 1.0000x reference)
import functools

import jax
import jax.numpy as jnp
from jax import lax
from jax.experimental import pallas as pl
from jax.experimental.pallas import tpu as pltpu
from jax.experimental.pallas import tpu_sc as plsc

N = 10000
H = 128
E = 320000

NC = 2
NS = 16
NW = NC * NS
CHUNK = 128
CPT = 80
E_PAD = NW * CPT * CHUNK
TOT_CHUNKS = NW * CPT
N_PAD = 10112
NROW = N_PAD // H
RPT = N_PAD // NS
DUMMY = N_PAD - 1

_mesh = plsc.VectorSubcoreMesh(core_axis_name="c", subcore_axis_name="s")
_f32 = jnp.float32
_sc_params = pltpu.CompilerParams(needs_layout_passes=False)


def _copy_row_to(srcref, j, dstref, width=CHUNK):
    for g in range(width // 16):
        sl = pl.ds(g * 16, 16)
        dstref[sl] = srcref[j, sl]


def _zero_1d(ref, n):
    @pl.loop(0, n, step=16)
    def _(i):
        ref[pl.ds(i, 16)] = jnp.zeros((16,), _f32)


@functools.partial(
    pl.kernel,
    out_type=jax.ShapeDtypeStruct((NW, N_PAD), _f32),
    mesh=_mesh,
    compiler_params=_sc_params,
    scratch_types=[
        pltpu.VMEM((CPT, CHUNK), jnp.int32),
        pltpu.VMEM((N_PAD,), _f32),
    ],
)
def _deg_pass(dst_hbm, out_hbm, dstbuf, acc):
    wid = lax.axis_index("s") * NC + lax.axis_index("c")
    _zero_1d(acc, N_PAD)
    pltpu.sync_copy(dst_hbm.at[pl.ds(wid * CPT, CPT)], dstbuf)
    ones = jnp.ones((16,), _f32)

    @pl.loop(0, CPT)
    def _(j):
        @pl.loop(0, CHUNK // 16)
        def _(g):
            didx = dstbuf[j, pl.ds(g * 16, 16)]
            plsc.addupdate_scatter(acc, [didx], ones)

    pltpu.sync_copy(acc, out_hbm.at[wid])


@functools.partial(
    pl.kernel,
    out_type=(
        jax.ShapeDtypeStruct((TOT_CHUNKS, CHUNK), _f32),
        jax.ShapeDtypeStruct((NW, N_PAD), _f32),
    ),
    mesh=_mesh,
    compiler_params=_sc_params,
    scratch_types=[
        pltpu.VMEM((CPT, CHUNK), jnp.int32),
        pltpu.VMEM((CPT, CHUNK), jnp.int32),
        pltpu.VMEM((N_PAD,), _f32),
        pltpu.VMEM((N_PAD,), _f32),
        pltpu.VMEM((CPT, CHUNK), _f32),
        pltpu.VMEM((N_PAD,), _f32),
    ],
)
def _edge_softmax(src_hbm, dst_hbm, ei_hbm, ej_hbm,
                  ex_hbm, s_hbm,
                  srcbuf, dstbuf, eibuf, ejbuf, exbuf, sacc):
    wid = lax.axis_index("s") * NC + lax.axis_index("c")
    _zero_1d(sacc, N_PAD)
    pltpu.sync_copy(src_hbm.at[pl.ds(wid * CPT, CPT)], srcbuf)
    pltpu.sync_copy(dst_hbm.at[pl.ds(wid * CPT, CPT)], dstbuf)
    pltpu.sync_copy(ei_hbm, eibuf)
    pltpu.sync_copy(ej_hbm, ejbuf)

    @pl.loop(0, CPT)
    def _(j):
        @pl.loop(0, CHUNK // 16)
        def _(g):
            sidx = srcbuf[j, pl.ds(g * 16, 16)]
            didx = dstbuf[j, pl.ds(g * 16, 16)]
            ev = plsc.load_gather(eibuf, [didx]) + plsc.load_gather(ejbuf, [sidx])
            ev = jnp.maximum(ev, 0.01 * ev)
            exv = jnp.exp(ev)
            exbuf[j, pl.ds(g * 16, 16)] = exv
            plsc.addupdate_scatter(sacc, [didx], exv)

    pltpu.sync_copy(exbuf, ex_hbm.at[pl.ds(wid * CPT, CPT)])
    pltpu.sync_copy(sacc, s_hbm.at[wid])


def _make_row_agg(weighted, SUB=64, NB=4):
    SPT = E_PAD // NW // SUB
    HSPT = SPT // 4
    scratch = (
        [pltpu.VMEM((HSPT, SUB), jnp.int32),
         pltpu.VMEM((HSPT, SUB), jnp.int32)]
        + ([pltpu.VMEM((HSPT, SUB), _f32)] if weighted else [])
        + ([pltpu.VMEM((SUB,), _f32)] * NB if weighted else [])
        + [pltpu.VMEM((SUB,), jnp.int32)] * NB
        + [pltpu.VMEM((SUB, H), _f32)] * NB
        + [pltpu.SemaphoreType.DMA] * NB
        + [pltpu.SemaphoreType.DMA] * NB
        + [pltpu.VMEM_SHARED((N_PAD, H), _f32)]
    )

    def body(*refs):
        if weighted:
            (y_hbm, src_hbm, dst_hbm, w_hbm, z_hbm, out_hbm,
             srcbuf, dstbuf, wbuf, *rest) = refs
            wrow = rest[:NB]
            rest = rest[NB:]
        else:
            (y_hbm, src_hbm, dst_hbm, z_hbm, out_hbm,
             srcbuf, dstbuf, *rest) = refs
        dstrow = rest[:NB]
        rows = rest[NB:2 * NB]
        gsem = rest[2 * NB:3 * NB]
        ssem = rest[3 * NB:4 * NB]
        acc = rest[4 * NB]
        cid = lax.axis_index("c")
        sid = lax.axis_index("s")
        wid = sid * NC + cid
        pltpu.sync_copy(z_hbm.at[pl.ds(sid * RPT, RPT)],
                        acc.at[pl.ds(sid * RPT, RPT)])
        plsc.subcore_barrier()

        for h in range(SPT // HSPT):
            pltpu.sync_copy(
                src_hbm.at[pl.ds(wid * SPT + h * HSPT, HSPT)], srcbuf)
            pltpu.sync_copy(
                dst_hbm.at[pl.ds(wid * SPT + h * HSPT, HSPT)], dstbuf)
            if weighted:
                pltpu.sync_copy(
                    w_hbm.at[pl.ds(wid * SPT + h * HSPT, HSPT)], wbuf)

            @pl.loop(0, HSPT, step=NB)
            def _(j):
                for b in range(NB):
                    @pl.when(jnp.logical_or(j >= NB, h > 0))
                    def _():
                        pltpu.make_async_copy(
                            z_hbm.at[pl.ds(0, SUB)], rows[b], ssem[b]).wait()
                    _copy_row_to(dstbuf, j + b, dstrow[b], SUB)
                    if weighted:
                        _copy_row_to(wbuf, j + b, wrow[b], SUB)
                    pltpu.async_copy(
                        y_hbm.at[srcbuf.at[j + b]], rows[b], gsem[b])
                for b in range(NB):
                    pltpu.make_async_copy(
                        z_hbm.at[pl.ds(0, SUB)], rows[b], gsem[b]).wait()
                    if weighted:
                        @pl.loop(0, SUB, unroll=2)
                        def _(i):
                            wv = plsc.load_gather(
                                wrow[b], [jnp.full((16,), i, jnp.int32)])
                            for cb in range(H // 16):
                                sl = pl.ds(cb * 16, 16)
                                rows[b][i, sl] = rows[b][i, sl] * wv
                    pltpu.async_copy(rows[b], acc.at[dstrow[b]], ssem[b],
                                     add=True)

        for b in range(NB):
            pltpu.make_async_copy(
                z_hbm.at[pl.ds(0, SUB)], rows[b], ssem[b]).wait()
        plsc.subcore_barrier()
        pltpu.sync_copy(acc.at[pl.ds(sid * RPT, RPT)],
                        out_hbm.at[cid].at[pl.ds(sid * RPT, RPT)])

    return pl.kernel(
        body,
        out_type=jax.ShapeDtypeStruct((NC, N_PAD, H), _f32),
        mesh=_mesh,
        compiler_params=_sc_params,
        scratch_types=scratch,
    )


SUB_RA = 64
NB_RA = 4
_row_agg = _make_row_agg(weighted=False, SUB=SUB_RA, NB=NB_RA)
_row_agg_w = _make_row_agg(weighted=True, SUB=SUB_RA, NB=NB_RA)


_tc_params = pltpu.CompilerParams(vmem_limit_bytes=100 * 1024 * 1024)


def _tc_dis_body(degp_ref, dis_ref):
    deg = jnp.sum(degp_ref[...], axis=0)
    dis_ref[...] = jnp.where(deg > 0, lax.rsqrt(jnp.maximum(deg, 1.0)), 0.0)


_tc_dis = pl.pallas_call(
    _tc_dis_body,
    out_shape=jax.ShapeDtypeStruct((NROW, H), _f32),
    compiler_params=_tc_params,
)


def _tc_ssum_body(sp_ref, s_ref):
    s_ref[...] = jnp.sum(sp_ref[...], axis=0)


_tc_ssum = pl.pallas_call(
    _tc_ssum_body,
    out_shape=jax.ShapeDtypeStruct((NROW, H), _f32),
    compiler_params=_tc_params,
)


def _tc_scale_body(x_ref, c_ref, y_ref):
    y_ref[...] = x_ref[...] * c_ref[...]


_tc_scale = pl.pallas_call(
    _tc_scale_body,
    out_shape=jax.ShapeDtypeStruct((N_PAD, H), _f32),
    compiler_params=_tc_params,
)


def _tc_layer_body(xp_ref, accp_ref, dis_ref, wt_ref, b_ref, xn_ref, yn_ref):
    dis = dis_ref[...]
    agg = jnp.maximum((accp_ref[0] + accp_ref[1]) * dis, 0.0)
    gate = jax.nn.sigmoid(
        jnp.dot(xp_ref[...], wt_ref[...], preferred_element_type=_f32)
        + b_ref[...])
    xn = gate * agg + (1.0 - gate) * xp_ref[...]
    xn_ref[...] = xn
    yn_ref[...] = xn * dis


_tc_layer = pl.pallas_call(
    _tc_layer_body,
    out_shape=(
        jax.ShapeDtypeStruct((N_PAD, H), _f32),
        jax.ShapeDtypeStruct((N_PAD, H), _f32),
    ),
    compiler_params=_tc_params,
)


def _tc_gatprep_body(x2_ref, x4_ref, a_ref, b_ref, eij_ref):
    eij_ref[...] = (
        jnp.dot(x2_ref[...], a_ref[...], preferred_element_type=_f32)
        + jnp.dot(x4_ref[...], b_ref[...], preferred_element_type=_f32))


_tc_gatprep = pl.pallas_call(
    _tc_gatprep_body,
    out_shape=jax.ShapeDtypeStruct((N_PAD, 2), _f32),
    compiler_params=_tc_params,
)


def _tc_gatfinal_body(u2_ref, u4_ref, s_ref, out_ref):
    s = s_ref[...]
    inv = jnp.where(s > 0, 1.0 / s, 0.0)
    o2 = jnp.maximum((u2_ref[0] + u2_ref[1]) * inv, 0.0)
    o4 = jnp.maximum((u4_ref[0] + u4_ref[1]) * inv, 0.0)
    out_ref[...] = jnp.concatenate([o2, o4], axis=1)


_tc_gatfinal = pl.pallas_call(
    _tc_gatfinal_body,
    out_shape=jax.ShapeDtypeStruct((N_PAD, 2 * H), _f32),
    compiler_params=_tc_params,
)


def kernel(x_e, edge_index, rel, edge_index_all,
           W1, b1, W2, b2, W3, b3, W4, b4, a_i, a_j):
    pad = jnp.full((E_PAD - E,), DUMMY, jnp.int32)
    src_f = jnp.concatenate([edge_index_all[0], pad])
    dst_f = jnp.concatenate([edge_index_all[1], pad])
    src = src_f.reshape(TOT_CHUNKS, CHUNK)
    dst = dst_f.reshape(TOT_CHUNKS, CHUNK)
    src_s = src_f.reshape(E_PAD // SUB_RA, SUB_RA)
    dst_s = dst_f.reshape(E_PAD // SUB_RA, SUB_RA)
    x0 = jnp.zeros((N_PAD, H), _f32).at[:N].set(x_e)
    z128 = jnp.zeros((N_PAD, H), _f32)

    degp = _deg_pass(dst)
    dis = _tc_dis(degp.reshape(NW, NROW, H)).reshape(N_PAD, 1)
    y = _tc_scale(x0, dis)

    xp = x0
    xs = []
    for Wm, bm in ((W1, b1), (W2, b2), (W3, b3), (W4, b4)):
        accp = _row_agg(y, src_s, dst_s, z128)
        xp, y = _tc_layer(xp, accp, dis, Wm.T, bm.reshape(1, H))
        xs.append(xp)
    x2, x4 = xs[1], xs[3]

    A = jnp.stack([a_i[:H], a_j[:H]], axis=1)
    Bm = jnp.stack([a_i[H:], a_j[H:]], axis=1)
    eij = _tc_gatprep(x2, x4, A, Bm)
    ei = eij[:, 0] + 0.0
    ej = eij[:, 1] + 0.0

    ex2d, sp = _edge_softmax(src, dst, ei, ej)
    ex_s = ex2d.reshape(E_PAD // SUB_RA, SUB_RA)
    s = _tc_ssum(sp.reshape(NW, NROW, H)).reshape(N_PAD, 1)
    u2 = _row_agg_w(x2, src_s, dst_s, ex_s, z128)
    u4 = _row_agg_w(x4, src_s, dst_s, ex_s, z128)
    out = _tc_gatfinal(u2, u4, s)
    return out[:N]


# speedup vs baseline: 1.0151x; 1.0151x over previous
import functools

import jax
import jax.numpy as jnp
from jax import lax
from jax.experimental import pallas as pl
from jax.experimental.pallas import tpu as pltpu
from jax.experimental.pallas import tpu_sc as plsc

N = 10000
H = 128
E = 320000

NC = 2
NS = 16
NW = NC * NS
CHUNK = 128
CPT = 80
E_PAD = NW * CPT * CHUNK
TOT_CHUNKS = NW * CPT
N_PAD = 10112
NROW = N_PAD // H
RPT = N_PAD // NS
DUMMY = N_PAD - 1

_mesh = plsc.VectorSubcoreMesh(core_axis_name="c", subcore_axis_name="s")
_f32 = jnp.float32
_sc_params = pltpu.CompilerParams(needs_layout_passes=False)


def _copy_row_to(srcref, j, dstref, width=CHUNK):
    for g in range(width // 16):
        sl = pl.ds(g * 16, 16)
        dstref[sl] = srcref[j, sl]


def _zero_1d(ref, n):
    @pl.loop(0, n, step=16)
    def _(i):
        ref[pl.ds(i, 16)] = jnp.zeros((16,), _f32)


@functools.partial(
    pl.kernel,
    out_type=jax.ShapeDtypeStruct((NW, N_PAD), _f32),
    mesh=_mesh,
    compiler_params=_sc_params,
    scratch_types=[
        pltpu.VMEM((CPT, CHUNK), jnp.int32),
        pltpu.VMEM((N_PAD,), _f32),
    ],
)
def _deg_pass(dst_hbm, out_hbm, dstbuf, acc):
    wid = lax.axis_index("s") * NC + lax.axis_index("c")
    _zero_1d(acc, N_PAD)
    pltpu.sync_copy(dst_hbm.at[pl.ds(wid * CPT, CPT)], dstbuf)
    ones = jnp.ones((16,), _f32)

    @pl.loop(0, CPT)
    def _(j):
        @pl.loop(0, CHUNK // 16)
        def _(g):
            didx = dstbuf[j, pl.ds(g * 16, 16)]
            plsc.addupdate_scatter(acc, [didx], ones)

    pltpu.sync_copy(acc, out_hbm.at[wid])


@functools.partial(
    pl.kernel,
    out_type=(
        jax.ShapeDtypeStruct((TOT_CHUNKS, CHUNK), _f32),
        jax.ShapeDtypeStruct((NW, N_PAD), _f32),
    ),
    mesh=_mesh,
    compiler_params=_sc_params,
    scratch_types=[
        pltpu.VMEM((CPT, CHUNK), jnp.int32),
        pltpu.VMEM((CPT, CHUNK), jnp.int32),
        pltpu.VMEM((N_PAD,), _f32),
        pltpu.VMEM((N_PAD,), _f32),
        pltpu.VMEM((CPT, CHUNK), _f32),
        pltpu.VMEM((N_PAD,), _f32),
    ],
)
def _edge_softmax(src_hbm, dst_hbm, ei_hbm, ej_hbm,
                  ex_hbm, s_hbm,
                  srcbuf, dstbuf, eibuf, ejbuf, exbuf, sacc):
    wid = lax.axis_index("s") * NC + lax.axis_index("c")
    _zero_1d(sacc, N_PAD)
    pltpu.sync_copy(src_hbm.at[pl.ds(wid * CPT, CPT)], srcbuf)
    pltpu.sync_copy(dst_hbm.at[pl.ds(wid * CPT, CPT)], dstbuf)
    pltpu.sync_copy(ei_hbm, eibuf)
    pltpu.sync_copy(ej_hbm, ejbuf)

    @pl.loop(0, CPT)
    def _(j):
        @pl.loop(0, CHUNK // 16)
        def _(g):
            sidx = srcbuf[j, pl.ds(g * 16, 16)]
            didx = dstbuf[j, pl.ds(g * 16, 16)]
            ev = plsc.load_gather(eibuf, [didx]) + plsc.load_gather(ejbuf, [sidx])
            ev = jnp.maximum(ev, 0.01 * ev)
            exv = jnp.exp(ev)
            exbuf[j, pl.ds(g * 16, 16)] = exv
            plsc.addupdate_scatter(sacc, [didx], exv)

    pltpu.sync_copy(exbuf, ex_hbm.at[pl.ds(wid * CPT, CPT)])
    pltpu.sync_copy(sacc, s_hbm.at[wid])


def _make_row_agg(weighted, SUB=64, NB=4):
    SPT = E_PAD // NW // SUB
    HSPT = 40
    scratch = (
        [pltpu.VMEM((HSPT, SUB), jnp.int32),
         pltpu.VMEM((HSPT, SUB), jnp.int32)]
        + ([pltpu.VMEM((HSPT, SUB), _f32)] if weighted else [])
        + ([pltpu.VMEM((SUB,), _f32)] * NB if weighted else [])
        + [pltpu.VMEM((SUB,), jnp.int32)] * NB
        + [pltpu.VMEM((SUB, H), _f32)] * NB
        + [pltpu.SemaphoreType.DMA] * NB
        + [pltpu.SemaphoreType.DMA] * NB
        + [pltpu.VMEM_SHARED((N_PAD, H), _f32)]
    )

    def body(*refs):
        if weighted:
            (y_hbm, src_hbm, dst_hbm, w_hbm, z_hbm, out_hbm,
             srcbuf, dstbuf, wbuf, *rest) = refs
            wrow = rest[:NB]
            rest = rest[NB:]
        else:
            (y_hbm, src_hbm, dst_hbm, z_hbm, out_hbm,
             srcbuf, dstbuf, *rest) = refs
        dstrow = rest[:NB]
        rows = rest[NB:2 * NB]
        gsem = rest[2 * NB:3 * NB]
        ssem = rest[3 * NB:4 * NB]
        acc = rest[4 * NB]
        cid = lax.axis_index("c")
        sid = lax.axis_index("s")
        wid = sid * NC + cid
        pltpu.sync_copy(z_hbm.at[pl.ds(sid * RPT, RPT)],
                        acc.at[pl.ds(sid * RPT, RPT)])
        plsc.subcore_barrier()

        for h in range(SPT // HSPT):
            pltpu.sync_copy(
                src_hbm.at[pl.ds(wid * SPT + h * HSPT, HSPT)], srcbuf)
            pltpu.sync_copy(
                dst_hbm.at[pl.ds(wid * SPT + h * HSPT, HSPT)], dstbuf)
            if weighted:
                pltpu.sync_copy(
                    w_hbm.at[pl.ds(wid * SPT + h * HSPT, HSPT)], wbuf)

            @pl.loop(0, HSPT, step=NB)
            def _(j):
                for b in range(NB):
                    @pl.when(jnp.logical_or(j >= NB, h > 0))
                    def _():
                        pltpu.make_async_copy(
                            z_hbm.at[pl.ds(0, SUB)], rows[b], ssem[b]).wait()
                    _copy_row_to(dstbuf, j + b, dstrow[b], SUB)
                    if weighted:
                        _copy_row_to(wbuf, j + b, wrow[b], SUB)
                    pltpu.async_copy(
                        y_hbm.at[srcbuf.at[j + b]], rows[b], gsem[b])
                for b in range(NB):
                    pltpu.make_async_copy(
                        z_hbm.at[pl.ds(0, SUB)], rows[b], gsem[b]).wait()
                    if weighted:
                        @pl.loop(0, SUB, unroll=2)
                        def _(i):
                            wv = plsc.load_gather(
                                wrow[b], [jnp.full((16,), i, jnp.int32)])
                            for cb in range(H // 16):
                                sl = pl.ds(cb * 16, 16)
                                rows[b][i, sl] = rows[b][i, sl] * wv
                    pltpu.async_copy(rows[b], acc.at[dstrow[b]], ssem[b],
                                     add=True)

        for b in range(NB):
            pltpu.make_async_copy(
                z_hbm.at[pl.ds(0, SUB)], rows[b], ssem[b]).wait()
        plsc.subcore_barrier()
        pltpu.sync_copy(acc.at[pl.ds(sid * RPT, RPT)],
                        out_hbm.at[cid].at[pl.ds(sid * RPT, RPT)])

    return pl.kernel(
        body,
        out_type=jax.ShapeDtypeStruct((NC, N_PAD, H), _f32),
        mesh=_mesh,
        compiler_params=_sc_params,
        scratch_types=scratch,
    )


SUB_RA = 128
NB_RA = 2
_row_agg = _make_row_agg(weighted=False, SUB=SUB_RA, NB=NB_RA)
_row_agg_w = _make_row_agg(weighted=True, SUB=SUB_RA, NB=NB_RA)


_tc_params = pltpu.CompilerParams(vmem_limit_bytes=100 * 1024 * 1024)


def _tc_dis_body(degp_ref, dis_ref):
    deg = jnp.sum(degp_ref[...], axis=0)
    dis_ref[...] = jnp.where(deg > 0, lax.rsqrt(jnp.maximum(deg, 1.0)), 0.0)


_tc_dis = pl.pallas_call(
    _tc_dis_body,
    out_shape=jax.ShapeDtypeStruct((NROW, H), _f32),
    compiler_params=_tc_params,
)


def _tc_ssum_body(sp_ref, s_ref):
    s_ref[...] = jnp.sum(sp_ref[...], axis=0)


_tc_ssum = pl.pallas_call(
    _tc_ssum_body,
    out_shape=jax.ShapeDtypeStruct((NROW, H), _f32),
    compiler_params=_tc_params,
)


def _tc_scale_body(x_ref, c_ref, y_ref):
    y_ref[...] = x_ref[...] * c_ref[...]


_tc_scale = pl.pallas_call(
    _tc_scale_body,
    out_shape=jax.ShapeDtypeStruct((N_PAD, H), _f32),
    compiler_params=_tc_params,
)


def _tc_layer_body(xp_ref, accp_ref, dis_ref, wt_ref, b_ref, xn_ref, yn_ref):
    dis = dis_ref[...]
    agg = jnp.maximum((accp_ref[0] + accp_ref[1]) * dis, 0.0)
    gate = jax.nn.sigmoid(
        jnp.dot(xp_ref[...], wt_ref[...], preferred_element_type=_f32)
        + b_ref[...])
    xn = gate * agg + (1.0 - gate) * xp_ref[...]
    xn_ref[...] = xn
    yn_ref[...] = xn * dis


_tc_layer = pl.pallas_call(
    _tc_layer_body,
    out_shape=(
        jax.ShapeDtypeStruct((N_PAD, H), _f32),
        jax.ShapeDtypeStruct((N_PAD, H), _f32),
    ),
    compiler_params=_tc_params,
)


def _tc_gatprep_body(x2_ref, x4_ref, a_ref, b_ref, eij_ref):
    eij_ref[...] = (
        jnp.dot(x2_ref[...], a_ref[...], preferred_element_type=_f32)
        + jnp.dot(x4_ref[...], b_ref[...], preferred_element_type=_f32))


_tc_gatprep = pl.pallas_call(
    _tc_gatprep_body,
    out_shape=jax.ShapeDtypeStruct((N_PAD, 2), _f32),
    compiler_params=_tc_params,
)


def _tc_gatfinal_body(u2_ref, u4_ref, s_ref, out_ref):
    s = s_ref[...]
    inv = jnp.where(s > 0, 1.0 / s, 0.0)
    o2 = jnp.maximum((u2_ref[0] + u2_ref[1]) * inv, 0.0)
    o4 = jnp.maximum((u4_ref[0] + u4_ref[1]) * inv, 0.0)
    out_ref[...] = jnp.concatenate([o2, o4], axis=1)


_tc_gatfinal = pl.pallas_call(
    _tc_gatfinal_body,
    out_shape=jax.ShapeDtypeStruct((N_PAD, 2 * H), _f32),
    compiler_params=_tc_params,
)


def kernel(x_e, edge_index, rel, edge_index_all,
           W1, b1, W2, b2, W3, b3, W4, b4, a_i, a_j):
    pad = jnp.full((E_PAD - E,), DUMMY, jnp.int32)
    src_f = jnp.concatenate([edge_index_all[0], pad])
    dst_f = jnp.concatenate([edge_index_all[1], pad])
    src = src_f.reshape(TOT_CHUNKS, CHUNK)
    dst = dst_f.reshape(TOT_CHUNKS, CHUNK)
    src_s = src_f.reshape(E_PAD // SUB_RA, SUB_RA)
    dst_s = dst_f.reshape(E_PAD // SUB_RA, SUB_RA)
    x0 = jnp.zeros((N_PAD, H), _f32).at[:N].set(x_e)
    z128 = jnp.zeros((N_PAD, H), _f32)

    degp = _deg_pass(dst)
    dis = _tc_dis(degp.reshape(NW, NROW, H)).reshape(N_PAD, 1)
    y = _tc_scale(x0, dis)

    xp = x0
    xs = []
    for Wm, bm in ((W1, b1), (W2, b2), (W3, b3), (W4, b4)):
        accp = _row_agg(y, src_s, dst_s, z128)
        xp, y = _tc_layer(xp, accp, dis, Wm.T, bm.reshape(1, H))
        xs.append(xp)
    x2, x4 = xs[1], xs[3]

    A = jnp.stack([a_i[:H], a_j[:H]], axis=1)
    Bm = jnp.stack([a_i[H:], a_j[H:]], axis=1)
    eij = _tc_gatprep(x2, x4, A, Bm)
    ei = eij[:, 0] + 0.0
    ej = eij[:, 1] + 0.0

    ex2d, sp = _edge_softmax(src, dst, ei, ej)
    ex_s = ex2d.reshape(E_PAD // SUB_RA, SUB_RA)
    s = _tc_ssum(sp.reshape(NW, NROW, H)).reshape(N_PAD, 1)
    u2 = _row_agg_w(x2, src_s, dst_s, ex_s, z128)
    u4 = _row_agg_w(x4, src_s, dst_s, ex_s, z128)
    out = _tc_gatfinal(u2, u4, s)
    return out[:N]


# speedup vs baseline: 1.2609x; 1.2422x over previous
import functools

import jax
import jax.numpy as jnp
from jax import lax
from jax.experimental import pallas as pl
from jax.experimental.pallas import tpu as pltpu
from jax.experimental.pallas import tpu_sc as plsc

N = 10000
H = 128
E = 320000

NC = 2
NS = 16
NW = NC * NS
CHUNK = 128
CPT = 80
E_PAD = NW * CPT * CHUNK
TOT_CHUNKS = NW * CPT
N_PAD = 10112
NROW = N_PAD // H
RPT = N_PAD // NS
DUMMY = N_PAD - 1

_mesh = plsc.VectorSubcoreMesh(core_axis_name="c", subcore_axis_name="s")
_f32 = jnp.float32
_sc_params = pltpu.CompilerParams(needs_layout_passes=False)


def _copy_row_to(srcref, j, dstref, width=CHUNK):
    for g in range(width // 16):
        sl = pl.ds(g * 16, 16)
        dstref[sl] = srcref[j, sl]


def _zero_1d(ref, n):
    @pl.loop(0, n, step=16)
    def _(i):
        ref[pl.ds(i, 16)] = jnp.zeros((16,), _f32)


@functools.partial(
    pl.kernel,
    out_type=jax.ShapeDtypeStruct((NW, N_PAD), _f32),
    mesh=_mesh,
    compiler_params=_sc_params,
    scratch_types=[
        pltpu.VMEM((CPT, CHUNK), jnp.int32),
        pltpu.VMEM((N_PAD,), _f32),
    ],
)
def _deg_pass(dst_hbm, out_hbm, dstbuf, acc):
    wid = lax.axis_index("s") * NC + lax.axis_index("c")
    _zero_1d(acc, N_PAD)
    pltpu.sync_copy(dst_hbm.at[pl.ds(wid * CPT, CPT)], dstbuf)
    ones = jnp.ones((16,), _f32)

    @pl.loop(0, CPT)
    def _(j):
        @pl.loop(0, CHUNK // 16)
        def _(g):
            didx = dstbuf[j, pl.ds(g * 16, 16)]
            plsc.addupdate_scatter(acc, [didx], ones)

    pltpu.sync_copy(acc, out_hbm.at[wid])


@functools.partial(
    pl.kernel,
    out_type=(
        jax.ShapeDtypeStruct((TOT_CHUNKS, CHUNK), _f32),
        jax.ShapeDtypeStruct((NW, N_PAD), _f32),
    ),
    mesh=_mesh,
    compiler_params=_sc_params,
    scratch_types=[
        pltpu.VMEM((CPT, CHUNK), jnp.int32),
        pltpu.VMEM((CPT, CHUNK), jnp.int32),
        pltpu.VMEM((N_PAD,), _f32),
        pltpu.VMEM((N_PAD,), _f32),
        pltpu.VMEM((CPT, CHUNK), _f32),
        pltpu.VMEM((N_PAD,), _f32),
    ],
)
def _edge_softmax(src_hbm, dst_hbm, ei_hbm, ej_hbm,
                  ex_hbm, s_hbm,
                  srcbuf, dstbuf, eibuf, ejbuf, exbuf, sacc):
    wid = lax.axis_index("s") * NC + lax.axis_index("c")
    _zero_1d(sacc, N_PAD)
    pltpu.sync_copy(src_hbm.at[pl.ds(wid * CPT, CPT)], srcbuf)
    pltpu.sync_copy(dst_hbm.at[pl.ds(wid * CPT, CPT)], dstbuf)
    pltpu.sync_copy(ei_hbm, eibuf)
    pltpu.sync_copy(ej_hbm, ejbuf)

    @pl.loop(0, CPT)
    def _(j):
        @pl.loop(0, CHUNK // 16)
        def _(g):
            sidx = srcbuf[j, pl.ds(g * 16, 16)]
            didx = dstbuf[j, pl.ds(g * 16, 16)]
            ev = plsc.load_gather(eibuf, [didx]) + plsc.load_gather(ejbuf, [sidx])
            ev = jnp.maximum(ev, 0.01 * ev)
            exv = jnp.exp(ev)
            exbuf[j, pl.ds(g * 16, 16)] = exv
            plsc.addupdate_scatter(sacc, [didx], exv)

    pltpu.sync_copy(exbuf, ex_hbm.at[pl.ds(wid * CPT, CPT)])
    pltpu.sync_copy(sacc, s_hbm.at[wid])


def _make_row_agg(weighted, SUB=64, NB=4):
    SPT2 = 2 * (E_PAD // NW // SUB)
    HSPT = 40
    NHT = SPT2 // HSPT
    NH0 = _NH0
    scratch = (
        [pltpu.VMEM((HSPT, SUB), jnp.int32),
         pltpu.VMEM((HSPT, SUB), jnp.int32)]
        + ([pltpu.VMEM((HSPT, SUB), _f32)] if weighted else [])
        + ([pltpu.VMEM((SUB,), _f32)] * NB if weighted else [])
        + [pltpu.VMEM((SUB,), jnp.int32)] * NB
        + [pltpu.VMEM((SUB, H), _f32)] * NB
        + [pltpu.SemaphoreType.DMA] * NB
        + [pltpu.SemaphoreType.DMA] * NB
        + [pltpu.VMEM_SHARED((N_PAD, H), _f32)]
    )

    def body(*refs):
        if weighted:
            (y_hbm, src_hbm, dst_hbm, w_hbm, z_hbm, out_hbm,
             srcbuf, dstbuf, wbuf, *rest) = refs
            wrow = rest[:NB]
            rest = rest[NB:]
        else:
            (y_hbm, src_hbm, dst_hbm, z_hbm, out_hbm,
             srcbuf, dstbuf, *rest) = refs
        dstrow = rest[:NB]
        rows = rest[NB:2 * NB]
        gsem = rest[2 * NB:3 * NB]
        ssem = rest[3 * NB:4 * NB]
        acc = rest[4 * NB]
        cid = lax.axis_index("c")
        sid = lax.axis_index("s")
        wid = sid * NC + cid
        pltpu.sync_copy(z_hbm.at[pl.ds(sid * RPT, RPT)],
                        acc.at[pl.ds(sid * RPT, RPT)])
        plsc.subcore_barrier()

        base = sid * NHT + cid * NH0
        nblk = jnp.where(cid == 0, NH0, NHT - NH0)

        @pl.loop(0, nblk)
        def _(h):
            off = pl.multiple_of((base + h) * HSPT, 8)
            pltpu.sync_copy(src_hbm.at[pl.ds(off, HSPT)], srcbuf)
            pltpu.sync_copy(dst_hbm.at[pl.ds(off, HSPT)], dstbuf)
            if weighted:
                pltpu.sync_copy(w_hbm.at[pl.ds(off, HSPT)], wbuf)

            @pl.loop(0, HSPT, step=NB)
            def _(j):
                for b in range(NB):
                    @pl.when(jnp.logical_or(j >= NB, h > 0))
                    def _():
                        pltpu.make_async_copy(
                            z_hbm.at[pl.ds(0, SUB)], rows[b], ssem[b]).wait()
                    _copy_row_to(dstbuf, j + b, dstrow[b], SUB)
                    if weighted:
                        _copy_row_to(wbuf, j + b, wrow[b], SUB)
                    pltpu.async_copy(
                        y_hbm.at[srcbuf.at[j + b]], rows[b], gsem[b])
                for b in range(NB):
                    pltpu.make_async_copy(
                        z_hbm.at[pl.ds(0, SUB)], rows[b], gsem[b]).wait()
                    if weighted:
                        @pl.loop(0, SUB, unroll=2)
                        def _(i):
                            wv = plsc.load_gather(
                                wrow[b], [jnp.full((16,), i, jnp.int32)])
                            for cb in range(H // 16):
                                sl = pl.ds(cb * 16, 16)
                                rows[b][i, sl] = rows[b][i, sl] * wv
                    pltpu.async_copy(rows[b], acc.at[dstrow[b]], ssem[b],
                                     add=True)

        for b in range(NB):
            pltpu.make_async_copy(
                z_hbm.at[pl.ds(0, SUB)], rows[b], ssem[b]).wait()
        plsc.subcore_barrier()
        pltpu.sync_copy(acc.at[pl.ds(sid * RPT, RPT)],
                        out_hbm.at[cid].at[pl.ds(sid * RPT, RPT)])

    return pl.kernel(
        body,
        out_type=jax.ShapeDtypeStruct((NC, N_PAD, H), _f32),
        mesh=_mesh,
        compiler_params=_sc_params,
        scratch_types=scratch,
    )


SUB_RA = 128
NB_RA = 2
_NH0 = 3
_row_agg = _make_row_agg(weighted=False, SUB=SUB_RA, NB=NB_RA)
_row_agg_w = _make_row_agg(weighted=True, SUB=SUB_RA, NB=NB_RA)


_tc_params = pltpu.CompilerParams(vmem_limit_bytes=100 * 1024 * 1024)


def _tc_dis_body(degp_ref, dis_ref):
    deg = jnp.sum(degp_ref[...], axis=0)
    dis_ref[...] = jnp.where(deg > 0, lax.rsqrt(jnp.maximum(deg, 1.0)), 0.0)


_tc_dis = pl.pallas_call(
    _tc_dis_body,
    out_shape=jax.ShapeDtypeStruct((NROW, H), _f32),
    compiler_params=_tc_params,
)


def _tc_ssum_body(sp_ref, s_ref):
    s_ref[...] = jnp.sum(sp_ref[...], axis=0)


_tc_ssum = pl.pallas_call(
    _tc_ssum_body,
    out_shape=jax.ShapeDtypeStruct((NROW, H), _f32),
    compiler_params=_tc_params,
)


def _tc_scale_body(x_ref, c_ref, y_ref):
    y_ref[...] = x_ref[...] * c_ref[...]


_tc_scale = pl.pallas_call(
    _tc_scale_body,
    out_shape=jax.ShapeDtypeStruct((N_PAD, H), _f32),
    compiler_params=_tc_params,
)


def _tc_layer_body(xp_ref, accp_ref, dis_ref, wt_ref, b_ref, xn_ref, yn_ref):
    dis = dis_ref[...]
    agg = jnp.maximum((accp_ref[0] + accp_ref[1]) * dis, 0.0)
    gate = jax.nn.sigmoid(
        jnp.dot(xp_ref[...], wt_ref[...], preferred_element_type=_f32)
        + b_ref[...])
    xn = gate * agg + (1.0 - gate) * xp_ref[...]
    xn_ref[...] = xn
    yn_ref[...] = xn * dis


_tc_layer = pl.pallas_call(
    _tc_layer_body,
    out_shape=(
        jax.ShapeDtypeStruct((N_PAD, H), _f32),
        jax.ShapeDtypeStruct((N_PAD, H), _f32),
    ),
    compiler_params=_tc_params,
)


def _tc_gatprep_body(x2_ref, x4_ref, a_ref, b_ref, eij_ref):
    eij_ref[...] = (
        jnp.dot(x2_ref[...], a_ref[...], preferred_element_type=_f32)
        + jnp.dot(x4_ref[...], b_ref[...], preferred_element_type=_f32))


_tc_gatprep = pl.pallas_call(
    _tc_gatprep_body,
    out_shape=jax.ShapeDtypeStruct((N_PAD, 2), _f32),
    compiler_params=_tc_params,
)


def _tc_gatfinal_body(u2_ref, u4_ref, s_ref, out_ref):
    s = s_ref[...]
    inv = jnp.where(s > 0, 1.0 / s, 0.0)
    o2 = jnp.maximum((u2_ref[0] + u2_ref[1]) * inv, 0.0)
    o4 = jnp.maximum((u4_ref[0] + u4_ref[1]) * inv, 0.0)
    out_ref[...] = jnp.concatenate([o2, o4], axis=1)


_tc_gatfinal = pl.pallas_call(
    _tc_gatfinal_body,
    out_shape=jax.ShapeDtypeStruct((N_PAD, 2 * H), _f32),
    compiler_params=_tc_params,
)


def kernel(x_e, edge_index, rel, edge_index_all,
           W1, b1, W2, b2, W3, b3, W4, b4, a_i, a_j):
    pad = jnp.full((E_PAD - E,), DUMMY, jnp.int32)
    src_f = jnp.concatenate([edge_index_all[0], pad])
    dst_f = jnp.concatenate([edge_index_all[1], pad])
    src = src_f.reshape(TOT_CHUNKS, CHUNK)
    dst = dst_f.reshape(TOT_CHUNKS, CHUNK)
    src_s = src_f.reshape(E_PAD // SUB_RA, SUB_RA)
    dst_s = dst_f.reshape(E_PAD // SUB_RA, SUB_RA)
    x0 = jnp.zeros((N_PAD, H), _f32).at[:N].set(x_e)
    z128 = jnp.zeros((N_PAD, H), _f32)

    degp = _deg_pass(dst)
    dis = _tc_dis(degp.reshape(NW, NROW, H)).reshape(N_PAD, 1)
    y = _tc_scale(x0, dis)

    xp = x0
    xs = []
    for Wm, bm in ((W1, b1), (W2, b2), (W3, b3), (W4, b4)):
        accp = _row_agg(y, src_s, dst_s, z128)
        xp, y = _tc_layer(xp, accp, dis, Wm.T, bm.reshape(1, H))
        xs.append(xp)
    x2, x4 = xs[1], xs[3]

    A = jnp.stack([a_i[:H], a_j[:H]], axis=1)
    Bm = jnp.stack([a_i[H:], a_j[H:]], axis=1)
    eij = _tc_gatprep(x2, x4, A, Bm)
    ei = eij[:, 0] + 0.0
    ej = eij[:, 1] + 0.0

    ex2d, sp = _edge_softmax(src, dst, ei, ej)
    ex_s = ex2d.reshape(E_PAD // SUB_RA, SUB_RA)
    s = _tc_ssum(sp.reshape(NW, NROW, H)).reshape(N_PAD, 1)
    u2 = _row_agg_w(x2, src_s, dst_s, ex_s, z128)
    u4 = _row_agg_w(x4, src_s, dst_s, ex_s, z128)
    out = _tc_gatfinal(u2, u4, s)
    return out[:N]
